# Initial kernel scaffold; baseline (speedup 1.0000x reference)
import jax
import jax.numpy as jnp
from jax.experimental import pallas as pl


def kernel(I, Wi, Wr, A):
    raise NotImplementedError("write your pallas kernel here")


# speedup vs baseline: 4.3218x; 4.3218x over previous
import jax
import jax.numpy as jnp
from jax import lax
from jax.experimental import pallas as pl
from jax.experimental.pallas import tpu as pltpu

_TAU = 1.0
_TILE_B = 1024


def _fused_scorer_kernel(i_ref, wi_ref, wr_ref, a_ref, probs_ref, logits_ref):
    proj_a = lax.dot_general(
        a_ref[...], wr_ref[...], (((1,), (1,)), ((), ())),
        preferred_element_type=jnp.float32)
    proj_i = lax.dot_general(
        i_ref[...], wi_ref[...], (((1,), (1,)), ((), ())),
        preferred_element_type=jnp.float32)
    logits = lax.dot_general(
        proj_i, proj_a, (((1,), (1,)), ((), ())),
        preferred_element_type=jnp.float32)
    logits_ref[...] = logits

    k = logits.shape[1]
    cols = lax.broadcasted_iota(jnp.int32, logits.shape, 1)
    m1 = jnp.max(logits, axis=1, keepdims=True)
    idx1 = jnp.min(jnp.where(logits == m1, cols, k), axis=1, keepdims=True)
    l2 = jnp.where(cols == idx1, -jnp.inf, logits)
    m2 = jnp.max(l2, axis=1, keepdims=True)
    idx2 = jnp.min(jnp.where(l2 == m2, cols, k), axis=1, keepdims=True)
    mask = (cols == idx1) | (cols == idx2)
    e = jnp.exp((logits - m1) / _TAU)
    p = jnp.where(mask, e, 0.0)
    probs_ref[...] = p / jnp.sum(p, axis=1, keepdims=True)


def kernel(I, Wi, Wr, A):
    B, d_in = I.shape
    k = A.shape[0]
    out = pl.pallas_call(
        _fused_scorer_kernel,
        grid=(B // _TILE_B,),
        in_specs=[
            pl.BlockSpec((_TILE_B, d_in), lambda i: (i, 0)),
            pl.BlockSpec(Wi.shape, lambda i: (0, 0)),
            pl.BlockSpec(Wr.shape, lambda i: (0, 0)),
            pl.BlockSpec(A.shape, lambda i: (0, 0)),
        ],
        out_specs=[
            pl.BlockSpec((_TILE_B, k), lambda i: (i, 0)),
            pl.BlockSpec((_TILE_B, k), lambda i: (i, 0)),
        ],
        out_shape=[
            jax.ShapeDtypeStruct((B, k), jnp.float32),
            jax.ShapeDtypeStruct((B, k), jnp.float32),
        ],
        compiler_params=pltpu.CompilerParams(
            dimension_semantics=("parallel",)),
    )(I, Wi, Wr, A)
    return (out[0], out[1])


# speedup vs baseline: 4.6121x; 1.0672x over previous
import jax
import jax.numpy as jnp
from jax import lax
from jax.experimental import pallas as pl
from jax.experimental.pallas import tpu as pltpu

_TAU = 1.0
_TILE_B = 1024


def _fused_scorer_kernel(i_ref, wi_ref, wr_ref, a_ref, probs_ref, logits_ref):
    proj_a = lax.dot_general(
        a_ref[...], wr_ref[...], (((1,), (1,)), ((), ())),
        preferred_element_type=jnp.float32)
    proj_i = lax.dot_general(
        i_ref[...], wi_ref[...], (((1,), (1,)), ((), ())),
        preferred_element_type=jnp.float32)
    logits = lax.dot_general(
        proj_i, proj_a, (((1,), (1,)), ((), ())),
        preferred_element_type=jnp.float32)
    logits_ref[...] = logits

    k = logits.shape[1]
    lt = (lax.broadcasted_iota(jnp.int32, (k, k), 0)
          <= lax.broadcasted_iota(jnp.int32, (k, k), 1)).astype(jnp.float32)
    m1 = jnp.max(logits, axis=1, keepdims=True)
    is1 = (logits == m1).astype(jnp.float32)
    c1 = jnp.dot(is1, lt, preferred_element_type=jnp.float32)
    first1 = (is1 > 0.0) & (c1 == 1.0)
    l2 = jnp.where(first1, -jnp.inf, logits)
    m2 = jnp.max(l2, axis=1, keepdims=True)
    is2 = (l2 == m2).astype(jnp.float32)
    c2 = jnp.dot(is2, lt, preferred_element_type=jnp.float32)
    first2 = (is2 > 0.0) & (c2 == 1.0)
    r = jnp.exp((m2 - m1) / _TAU)
    denom = 1.0 + r
    probs_ref[...] = jnp.where(first1, 1.0 / denom,
                               jnp.where(first2, r / denom, 0.0))


def kernel(I, Wi, Wr, A):
    B, d_in = I.shape
    k = A.shape[0]
    out = pl.pallas_call(
        _fused_scorer_kernel,
        grid=(B // _TILE_B,),
        in_specs=[
            pl.BlockSpec((_TILE_B, d_in), lambda i: (i, 0)),
            pl.BlockSpec(Wi.shape, lambda i: (0, 0)),
            pl.BlockSpec(Wr.shape, lambda i: (0, 0)),
            pl.BlockSpec(A.shape, lambda i: (0, 0)),
        ],
        out_specs=[
            pl.BlockSpec((_TILE_B, k), lambda i: (i, 0)),
            pl.BlockSpec((_TILE_B, k), lambda i: (i, 0)),
        ],
        out_shape=[
            jax.ShapeDtypeStruct((B, k), jnp.float32),
            jax.ShapeDtypeStruct((B, k), jnp.float32),
        ],
        compiler_params=pltpu.CompilerParams(
            dimension_semantics=("parallel",)),
    )(I, Wi, Wr, A)
    return (out[0], out[1])


# speedup vs baseline: 4.6930x; 1.0175x over previous
import jax
import jax.numpy as jnp
from jax import lax
from jax.experimental import pallas as pl
from jax.experimental.pallas import tpu as pltpu

_TAU = 1.0
_TILE_B = 1024
_SPLIT = 4
_SUB = _TILE_B // _SPLIT


def _fused_scorer_kernel(i0_ref, i1_ref, i2_ref, i3_ref, wi_ref, wr_ref,
                         a_ref, probs_ref, logits_ref):
    proj_a = lax.dot_general(
        a_ref[...], wr_ref[...], (((1,), (1,)), ((), ())),
        preferred_element_type=jnp.float32)
    proj_i = jnp.concatenate(
        [lax.dot_general(ref[...], wi_ref[...], (((1,), (1,)), ((), ())),
                         preferred_element_type=jnp.float32)
         for ref in (i0_ref, i1_ref, i2_ref, i3_ref)],
        axis=0)
    logits = lax.dot_general(
        proj_i, proj_a, (((1,), (1,)), ((), ())),
        preferred_element_type=jnp.float32)
    logits_ref[...] = logits

    k = logits.shape[1]
    lt = (lax.broadcasted_iota(jnp.int32, (k, k), 0)
          <= lax.broadcasted_iota(jnp.int32, (k, k), 1)).astype(jnp.float32)
    m1 = jnp.max(logits, axis=1, keepdims=True)
    is1 = (logits == m1).astype(jnp.float32)
    c1 = jnp.dot(is1, lt, preferred_element_type=jnp.float32)
    first1 = (is1 > 0.0) & (c1 == 1.0)
    l2 = jnp.where(first1, -jnp.inf, logits)
    m2 = jnp.max(l2, axis=1, keepdims=True)
    is2 = (l2 == m2).astype(jnp.float32)
    c2 = jnp.dot(is2, lt, preferred_element_type=jnp.float32)
    first2 = (is2 > 0.0) & (c2 == 1.0)
    r = jnp.exp((m2 - m1) / _TAU)
    denom = 1.0 + r
    probs_ref[...] = jnp.where(first1, 1.0 / denom,
                               jnp.where(first2, r / denom, 0.0))


def _i_spec(j):
    return pl.BlockSpec((_SUB, 768), lambda i, j=j: (_SPLIT * i + j, 0))


def kernel(I, Wi, Wr, A):
    B, d_in = I.shape
    k = A.shape[0]
    out = pl.pallas_call(
        _fused_scorer_kernel,
        grid=(B // _TILE_B,),
        in_specs=[
            _i_spec(0), _i_spec(1), _i_spec(2), _i_spec(3),
            pl.BlockSpec(Wi.shape, lambda i: (0, 0)),
            pl.BlockSpec(Wr.shape, lambda i: (0, 0)),
            pl.BlockSpec(A.shape, lambda i: (0, 0)),
        ],
        out_specs=[
            pl.BlockSpec((_TILE_B, k), lambda i: (i, 0)),
            pl.BlockSpec((_TILE_B, k), lambda i: (i, 0)),
        ],
        out_shape=[
            jax.ShapeDtypeStruct((B, k), jnp.float32),
            jax.ShapeDtypeStruct((B, k), jnp.float32),
        ],
        compiler_params=pltpu.CompilerParams(
            dimension_semantics=("parallel",)),
    )(I, I, I, I, Wi, Wr, A)
    return (out[0], out[1])


# speedup vs baseline: 4.7157x; 1.0048x over previous
import jax
import jax.numpy as jnp
from jax import lax
from jax.experimental import pallas as pl
from jax.experimental.pallas import tpu as pltpu

_TAU = 1.0
_TILE_B = 1024
_SPLIT = 8
_SUB = _TILE_B // _SPLIT


def _fused_scorer_kernel(*refs):
    (*i_refs, wi_ref, wr_ref, a_ref, probs_ref, logits_ref) = refs
    proj_a = lax.dot_general(
        a_ref[...], wr_ref[...], (((1,), (1,)), ((), ())),
        preferred_element_type=jnp.float32)
    proj_i = jnp.concatenate(
        [lax.dot_general(ref[...], wi_ref[...], (((1,), (1,)), ((), ())),
                         preferred_element_type=jnp.float32)
         for ref in i_refs],
        axis=0)
    logits = lax.dot_general(
        proj_i, proj_a, (((1,), (1,)), ((), ())),
        preferred_element_type=jnp.float32)
    logits_ref[...] = logits

    k = logits.shape[1]
    lt = (lax.broadcasted_iota(jnp.int32, (k, k), 0)
          <= lax.broadcasted_iota(jnp.int32, (k, k), 1)).astype(jnp.float32)
    m1 = jnp.max(logits, axis=1, keepdims=True)
    is1 = (logits == m1).astype(jnp.float32)
    c1 = jnp.dot(is1, lt, preferred_element_type=jnp.float32)
    first1 = (is1 > 0.0) & (c1 == 1.0)
    l2 = jnp.where(first1, -jnp.inf, logits)
    m2 = jnp.max(l2, axis=1, keepdims=True)
    is2 = (l2 == m2).astype(jnp.float32)
    c2 = jnp.dot(is2, lt, preferred_element_type=jnp.float32)
    first2 = (is2 > 0.0) & (c2 == 1.0)
    r = jnp.exp((m2 - m1) / _TAU)
    denom = 1.0 + r
    probs_ref[...] = jnp.where(first1, 1.0 / denom,
                               jnp.where(first2, r / denom, 0.0))


def _i_spec(j):
    return pl.BlockSpec((_SUB, 768), lambda i, j=j: (_SPLIT * i + j, 0))


def kernel(I, Wi, Wr, A):
    B, d_in = I.shape
    k = A.shape[0]
    out = pl.pallas_call(
        _fused_scorer_kernel,
        grid=(B // _TILE_B,),
        in_specs=[
            *[_i_spec(j) for j in range(_SPLIT)],
            pl.BlockSpec(Wi.shape, lambda i: (0, 0)),
            pl.BlockSpec(Wr.shape, lambda i: (0, 0)),
            pl.BlockSpec(A.shape, lambda i: (0, 0)),
        ],
        out_specs=[
            pl.BlockSpec((_TILE_B, k), lambda i: (i, 0)),
            pl.BlockSpec((_TILE_B, k), lambda i: (i, 0)),
        ],
        out_shape=[
            jax.ShapeDtypeStruct((B, k), jnp.float32),
            jax.ShapeDtypeStruct((B, k), jnp.float32),
        ],
        compiler_params=pltpu.CompilerParams(
            dimension_semantics=("parallel",)),
    )(*([I] * _SPLIT), Wi, Wr, A)
    return (out[0], out[1])


# speedup vs baseline: 5.3441x; 1.1333x over previous
import jax
import jax.numpy as jnp
from jax import lax
from jax.experimental import pallas as pl
from jax.experimental.pallas import tpu as pltpu

_TAU = 1.0
_TILE_B = 2048
_SPLIT = 8
_SUB = _TILE_B // _SPLIT


def _fused_scorer_kernel(*refs):
    (*i_refs, wi_ref, wr_ref, a_ref, probs_ref, logits_ref) = refs
    proj_a = lax.dot_general(
        a_ref[...], wr_ref[...], (((1,), (1,)), ((), ())),
        preferred_element_type=jnp.float32)
    proj_i = jnp.concatenate(
        [lax.dot_general(ref[...], wi_ref[...], (((1,), (1,)), ((), ())),
                         preferred_element_type=jnp.float32)
         for ref in i_refs],
        axis=0)
    logits = lax.dot_general(
        proj_i, proj_a, (((1,), (1,)), ((), ())),
        preferred_element_type=jnp.float32)
    logits_ref[...] = logits

    k = logits.shape[1]
    lt = (lax.broadcasted_iota(jnp.int32, (k, k), 0)
          <= lax.broadcasted_iota(jnp.int32, (k, k), 1)).astype(jnp.float32)
    m1 = jnp.max(logits, axis=1, keepdims=True)
    is1 = (logits == m1).astype(jnp.float32)
    c1 = jnp.dot(is1, lt, preferred_element_type=jnp.float32)
    first1 = (is1 > 0.0) & (c1 == 1.0)
    l2 = jnp.where(first1, -jnp.inf, logits)
    m2 = jnp.max(l2, axis=1, keepdims=True)
    is2 = (l2 == m2).astype(jnp.float32)
    c2 = jnp.dot(is2, lt, preferred_element_type=jnp.float32)
    first2 = (is2 > 0.0) & (c2 == 1.0)
    r = jnp.exp((m2 - m1) / _TAU)
    denom = 1.0 + r
    probs_ref[...] = jnp.where(first1, 1.0 / denom,
                               jnp.where(first2, r / denom, 0.0))


def _i_spec(j):
    return pl.BlockSpec((_SUB, 768), lambda i, j=j: (_SPLIT * i + j, 0))


def kernel(I, Wi, Wr, A):
    B, d_in = I.shape
    k = A.shape[0]
    out = pl.pallas_call(
        _fused_scorer_kernel,
        grid=(B // _TILE_B,),
        in_specs=[
            *[_i_spec(j) for j in range(_SPLIT)],
            pl.BlockSpec(Wi.shape, lambda i: (0, 0)),
            pl.BlockSpec(Wr.shape, lambda i: (0, 0)),
            pl.BlockSpec(A.shape, lambda i: (0, 0)),
        ],
        out_specs=[
            pl.BlockSpec((_TILE_B, k), lambda i: (i, 0)),
            pl.BlockSpec((_TILE_B, k), lambda i: (i, 0)),
        ],
        out_shape=[
            jax.ShapeDtypeStruct((B, k), jnp.float32),
            jax.ShapeDtypeStruct((B, k), jnp.float32),
        ],
        compiler_params=pltpu.CompilerParams(
            dimension_semantics=("parallel",)),
    )(*([I] * _SPLIT), Wi, Wr, A)
    return (out[0], out[1])


# speedup vs baseline: 5.5151x; 1.0320x over previous
import jax
import jax.numpy as jnp
from jax import lax
from jax.experimental import pallas as pl
from jax.experimental.pallas import tpu as pltpu

_TAU = 1.0
_TILE_B = 4096
_SPLIT = 16
_SUB = _TILE_B // _SPLIT


def _fused_scorer_kernel(*refs):
    (*i_refs, wi_ref, wr_ref, a_ref, probs_ref, logits_ref) = refs
    proj_a = lax.dot_general(
        a_ref[...], wr_ref[...], (((1,), (1,)), ((), ())),
        preferred_element_type=jnp.float32)
    proj_i = jnp.concatenate(
        [lax.dot_general(ref[...], wi_ref[...], (((1,), (1,)), ((), ())),
                         preferred_element_type=jnp.float32)
         for ref in i_refs],
        axis=0)
    logits = lax.dot_general(
        proj_i, proj_a, (((1,), (1,)), ((), ())),
        preferred_element_type=jnp.float32)
    logits_ref[...] = logits

    k = logits.shape[1]
    lt = (lax.broadcasted_iota(jnp.int32, (k, k), 0)
          <= lax.broadcasted_iota(jnp.int32, (k, k), 1)).astype(jnp.float32)
    m1 = jnp.max(logits, axis=1, keepdims=True)
    is1 = (logits == m1).astype(jnp.float32)
    c1 = jnp.dot(is1, lt, preferred_element_type=jnp.float32)
    first1 = (is1 > 0.0) & (c1 == 1.0)
    l2 = jnp.where(first1, -jnp.inf, logits)
    m2 = jnp.max(l2, axis=1, keepdims=True)
    is2 = (l2 == m2).astype(jnp.float32)
    c2 = jnp.dot(is2, lt, preferred_element_type=jnp.float32)
    first2 = (is2 > 0.0) & (c2 == 1.0)
    r = jnp.exp((m2 - m1) / _TAU)
    denom = 1.0 + r
    probs_ref[...] = jnp.where(first1, 1.0 / denom,
                               jnp.where(first2, r / denom, 0.0))


def _i_spec(j):
    return pl.BlockSpec((_SUB, 768), lambda i, j=j: (_SPLIT * i + j, 0))


def kernel(I, Wi, Wr, A):
    B, d_in = I.shape
    k = A.shape[0]
    out = pl.pallas_call(
        _fused_scorer_kernel,
        grid=(B // _TILE_B,),
        in_specs=[
            *[_i_spec(j) for j in range(_SPLIT)],
            pl.BlockSpec(Wi.shape, lambda i: (0, 0)),
            pl.BlockSpec(Wr.shape, lambda i: (0, 0)),
            pl.BlockSpec(A.shape, lambda i: (0, 0)),
        ],
        out_specs=[
            pl.BlockSpec((_TILE_B, k), lambda i: (i, 0)),
            pl.BlockSpec((_TILE_B, k), lambda i: (i, 0)),
        ],
        out_shape=[
            jax.ShapeDtypeStruct((B, k), jnp.float32),
            jax.ShapeDtypeStruct((B, k), jnp.float32),
        ],
        compiler_params=pltpu.CompilerParams(
            dimension_semantics=("parallel",)),
    )(*([I] * _SPLIT), Wi, Wr, A)
    return (out[0], out[1])
